# Initial kernel scaffold; baseline (speedup 1.0000x reference)
import jax
import jax.numpy as jnp
from jax.experimental import pallas as pl


def kernel(x, enc0_w, enc0_b, enc0_g, enc0_be, enc1_w, enc1_b, enc1_g, enc1_be, prevq_w, prevq_b, emb, dec0_w, dec0_b, dec0_g, dec0_be, dec1_w, dec1_b, dec1_g, dec1_be, proj_w, proj_b, ro_w, ro_b):
    raise NotImplementedError("write your pallas kernel here")


# speedup vs baseline: 3.3699x; 3.3699x over previous
import functools

import jax
import jax.numpy as jnp
from jax.experimental import pallas as pl
from jax.experimental.pallas import tpu as pltpu


def _conv2d(x, w, b, stride, pad):
    out = jax.lax.conv_general_dilated(
        x, w, (stride, stride), [(pad, pad), (pad, pad)],
        dimension_numbers=('NCHW', 'OIHW', 'NCHW'))
    return out + b[None, :, None, None]


def _conv_t2d(x, w, b, stride, pad, out_pad):
    wt = jnp.flip(jnp.transpose(w, (1, 0, 2, 3)), axis=(2, 3))
    k = w.shape[2]
    out = jax.lax.conv_general_dilated(
        x, wt, (1, 1), [(k - 1 - pad, k - 1 - pad + out_pad)] * 2,
        lhs_dilation=(stride, stride), dimension_numbers=('NCHW', 'OIHW', 'NCHW'))
    return out + b[None, :, None, None]


def _group_norm(x, g, bta, groups=2, eps=1e-5):
    B, C, H, W = x.shape
    xg = x.reshape(B, groups, C // groups, H, W)
    m = xg.mean(axis=(2, 3, 4), keepdims=True)
    v = ((xg - m) ** 2).mean(axis=(2, 3, 4), keepdims=True)
    xn = ((xg - m) / jnp.sqrt(v + eps)).reshape(B, C, H, W)
    return xn * g[None, :, None, None] + bta[None, :, None, None]


def _lrelu(x):
    return jnp.where(x >= 0, x, 0.2 * x)


def _vq_body(flat_ref, emb_ref, quant_ref, loss_ref, perp_ref,
             counts_acc, loss_acc, *, n_rows, n_steps, commitment_cost):
    step = pl.program_id(0)
    flat = flat_ref[...]
    emb = emb_ref[...]
    R = flat.shape[0]
    K = emb.shape[0]

    s = 2.0 * jax.lax.dot_general(
        flat, emb, (((1,), (1,)), ((), ())),
        preferred_element_type=jnp.float32)
    s = s - jnp.sum(emb * emb, axis=1)[None, :]

    lane = jax.lax.broadcasted_iota(jnp.int32, (R, K), 1)

    def pick(sc):
        m = jnp.max(sc, axis=1, keepdims=True)
        i = jnp.min(jnp.where(sc == m, lane, K), axis=1, keepdims=True)
        return m, i

    _, i0 = pick(s)
    s = jnp.where(lane == i0, -jnp.inf, s)
    _, i1 = pick(s)
    s = jnp.where(lane == i1, -jnp.inf, s)
    _, i2 = pick(s)

    oh0 = (lane == i0).astype(jnp.float32)
    quant = jax.lax.dot_general(
        oh0, emb, (((1,), (0,)), ((), ())),
        preferred_element_type=jnp.float32)
    quant_ref[...] = quant

    resid = flat - quant
    block_loss = jnp.sum(resid * resid)
    block_counts = jnp.sum((lane == i2).astype(jnp.float32), axis=0)[None, :]

    @pl.when(step == 0)
    def _init():
        loss_acc[...] = jnp.zeros_like(loss_acc)
        counts_acc[...] = jnp.zeros_like(counts_acc)

    loss_acc[...] += block_loss.reshape(1, 1)
    counts_acc[...] += block_counts

    @pl.when(step == n_steps - 1)
    def _fin():
        n_elems = jnp.float32(n_rows) * jnp.float32(flat.shape[1])
        loss_ref[...] = commitment_cost * loss_acc[...] / n_elems
        p = counts_acc[...] / jnp.float32(n_rows)
        ent = -jnp.sum(p * jnp.log(p + 1e-10))
        perp_ref[...] = jnp.exp(ent).reshape(1, 1)


def _vq_stage(flat, emb, commitment_cost):
    n_rows, ed = flat.shape
    K = emb.shape[0]
    R = 512
    n_steps = n_rows // R
    assert n_steps * R == n_rows

    body = functools.partial(_vq_body, n_rows=n_rows, n_steps=n_steps,
                             commitment_cost=commitment_cost)
    quant, loss, perp = pl.pallas_call(
        body,
        grid=(n_steps,),
        in_specs=[
            pl.BlockSpec((R, ed), lambda i: (i, 0)),
            pl.BlockSpec((K, ed), lambda i: (0, 0)),
        ],
        out_specs=[
            pl.BlockSpec((R, ed), lambda i: (i, 0)),
            pl.BlockSpec((1, 1), lambda i: (0, 0)),
            pl.BlockSpec((1, 1), lambda i: (0, 0)),
        ],
        out_shape=[
            jax.ShapeDtypeStruct((n_rows, ed), jnp.float32),
            jax.ShapeDtypeStruct((1, 1), jnp.float32),
            jax.ShapeDtypeStruct((1, 1), jnp.float32),
        ],
        scratch_shapes=[
            pltpu.VMEM((1, K), jnp.float32),
            pltpu.VMEM((1, 1), jnp.float32),
        ],
    )(flat, emb)
    return quant, loss[0, 0], perp[0, 0]


def kernel(x, enc0_w, enc0_b, enc0_g, enc0_be, enc1_w, enc1_b, enc1_g,
           enc1_be, prevq_w, prevq_b, emb, dec0_w, dec0_b, dec0_g, dec0_be,
           dec1_w, dec1_b, dec1_g, dec1_be, proj_w, proj_b, ro_w, ro_b):
    commitment_cost = 0.25
    enc1 = _lrelu(_group_norm(_conv2d(x, enc0_w, enc0_b, 1, 1), enc0_g, enc0_be))
    latent = _lrelu(_group_norm(_conv2d(enc1, enc1_w, enc1_b, 2, 1), enc1_g, enc1_be))
    z = _conv2d(latent, prevq_w, prevq_b, 1, 0)
    inputs = jnp.transpose(z, (0, 2, 3, 1))
    flat = inputs.reshape(-1, inputs.shape[-1])
    quant_flat, loss, perplexity = _vq_stage(flat, emb, commitment_cost)
    qz = jnp.transpose(quant_flat.reshape(inputs.shape), (0, 3, 1, 2))
    skip_p = _conv2d(enc1, proj_w, proj_b, 1, 0)
    hid = _lrelu(_group_norm(_conv_t2d(qz, dec0_w, dec0_b, 2, 1, 1), dec0_g, dec0_be))
    cat = jnp.concatenate([hid, skip_p], axis=1)
    y = _lrelu(_group_norm(_conv2d(cat, dec1_w, dec1_b, 1, 1), dec1_g, dec1_be))
    x_recon = _conv2d(y, ro_w, ro_b, 1, 0)
    return (loss, x_recon, perplexity)
